# Initial kernel scaffold; baseline (speedup 1.0000x reference)
import jax
import jax.numpy as jnp
from jax.experimental import pallas as pl


def kernel(x, edge_index, batch, W_l1, b_l1, W_r1, W_l2, b_l2, W_r2, W_out, b_out):
    raise NotImplementedError("write your pallas kernel here")


# speedup vs baseline: 3.6112x; 3.6112x over previous
import functools

import jax
import jax.numpy as jnp
from jax import lax
from jax.experimental import pallas as pl
from jax.experimental.pallas import tpu as pltpu
from jax.experimental.pallas import tpu_sc as plsc

N_NODES = 10000
FDIM = 256
HALF = 128
NGRAPH = 64

NC = 2
NS = 16
CHUNK = 128
SUPER = 8

ZROWS = -(-(N_NODES + 1) // (NS * 8)) * 8
N_ACC = ZROWS * NS


def _sc_aggregate_kernel(nchunks, xcat, src_r, dst_r, zeros_rows,
                         agg_out, acc, src_v, dst_v, rows_v, gsem):
  c = lax.axis_index("c")
  s = lax.axis_index("s")

  pltpu.sync_copy(zeros_rows, acc.at[pl.ds(s * ZROWS, ZROWS)])
  plsc.subcore_barrier()

  def outer(u, carry):
    pltpu.sync_copy(src_r.at[c, s, pl.ds(u * SUPER, SUPER)], src_v)
    pltpu.sync_copy(dst_r.at[c, s, pl.ds(u * SUPER, SUPER)], dst_v)
    for j in range(SUPER):
      pltpu.async_copy(xcat.at[src_v.at[j]], rows_v, gsem).wait()
      pltpu.sync_copy(rows_v, acc.at[dst_v.at[j]], add=True)
    return carry

  lax.fori_loop(0, nchunks // SUPER, outer, 0)
  plsc.subcore_barrier()

  pltpu.sync_copy(acc.at[pl.ds(s * ZROWS, ZROWS)],
                  agg_out.at[pl.ds(c * N_ACC + s * ZROWS, ZROWS)])


def _make_sc_aggregate(nchunks):
  mesh = plsc.VectorSubcoreMesh(core_axis_name="c", subcore_axis_name="s")
  scratch = [
      pltpu.VMEM_SHARED((N_ACC, HALF), jnp.float32),
      pltpu.VMEM((SUPER, CHUNK), jnp.int32),
      pltpu.VMEM((SUPER, CHUNK), jnp.int32),
      pltpu.VMEM((CHUNK, HALF), jnp.float32),
      pltpu.SemaphoreType.DMA,
  ]
  return pl.kernel(
      functools.partial(_sc_aggregate_kernel, nchunks),
      out_type=jax.ShapeDtypeStruct((NC * N_ACC, HALF), jnp.float32),
      mesh=mesh, scratch_types=scratch)


def _sc_count_kernel(nchunks, dst_r, zeros_rows, ones_rows,
                     cnt_out, acc, dst_v, ones_v, gsem):
  c = lax.axis_index("c")
  s = lax.axis_index("s")

  pltpu.sync_copy(zeros_rows, acc.at[pl.ds(s * ZROWS, ZROWS)])
  pltpu.sync_copy(ones_rows, ones_v)
  plsc.subcore_barrier()

  def outer(u, carry):
    pltpu.sync_copy(dst_r.at[c, s, pl.ds(u * SUPER, SUPER)], dst_v)
    for j in range(SUPER):
      pltpu.sync_copy(ones_v, acc.at[dst_v.at[j]], add=True)
    return carry

  lax.fori_loop(0, nchunks // SUPER, outer, 0)
  plsc.subcore_barrier()

  pltpu.sync_copy(acc.at[pl.ds(s * ZROWS, ZROWS)],
                  cnt_out.at[pl.ds(c * N_ACC + s * ZROWS, ZROWS)])


def _make_sc_count(nchunks):
  mesh = plsc.VectorSubcoreMesh(core_axis_name="c", subcore_axis_name="s")
  scratch = [
      pltpu.VMEM_SHARED((N_ACC, HALF), jnp.float32),
      pltpu.VMEM((SUPER, CHUNK), jnp.int32),
      pltpu.VMEM((CHUNK, HALF), jnp.float32),
      pltpu.SemaphoreType.DMA,
  ]
  return pl.kernel(
      functools.partial(_sc_count_kernel, nchunks),
      out_type=jax.ShapeDtypeStruct((NC * N_ACC, HALF), jnp.float32),
      mesh=mesh, scratch_types=scratch)


def _sage_tc_kernel(agg_lo, agg_hi, x_lo, x_hi, cnt0, cnt1, w_l, b_l, w_r,
                    h_out):
  inv = 1.0 / jnp.maximum(cnt0[0, :, 0:1] + cnt1[0, :, 0:1], 1.0)
  acc = jnp.dot(agg_lo[0], w_l[:HALF, :], preferred_element_type=jnp.float32)
  acc += jnp.dot(agg_hi[0], w_l[HALF:, :], preferred_element_type=jnp.float32)
  acc *= inv
  acc += jnp.dot(x_lo[...], w_r[:HALF, :], preferred_element_type=jnp.float32)
  acc += jnp.dot(x_hi[...], w_r[HALF:, :], preferred_element_type=jnp.float32)
  acc += b_l[...]
  h = jnp.maximum(acc, 0.0)
  h_out[0] = h[:, :HALF]
  h_out[1] = h[:, HALF:]


def _sage_pool_tc_kernel(nb, blk, agg_lo, agg_hi, x_lo, x_hi, cnt0, cnt1,
                         w_l, b_l, w_r, batch, w_out, b_out, out,
                         psum, pmax, pcnt):
  i = pl.program_id(0)

  @pl.when(i == 0)
  def _():
    psum[...] = jnp.zeros_like(psum)
    pmax[...] = jnp.full_like(pmax, -1e30)
    pcnt[...] = jnp.zeros_like(pcnt)

  inv = 1.0 / jnp.maximum(cnt0[0, :, 0:1] + cnt1[0, :, 0:1], 1.0)
  acc = jnp.dot(agg_lo[0], w_l[:HALF, :], preferred_element_type=jnp.float32)
  acc += jnp.dot(agg_hi[0], w_l[HALF:, :], preferred_element_type=jnp.float32)
  acc *= inv
  acc += jnp.dot(x_lo[0], w_r[:HALF, :], preferred_element_type=jnp.float32)
  acc += jnp.dot(x_hi[0], w_r[HALF:, :], preferred_element_type=jnp.float32)
  acc += b_l[...]
  h = jnp.maximum(acc, 0.0)

  ids = batch[0]
  onehot = (ids == lax.broadcasted_iota(jnp.int32, (blk, NGRAPH), 1)
            ).astype(jnp.float32)
  psum[...] += lax.dot_general(onehot, h, (((0,), (0,)), ((), ())),
                               preferred_element_type=jnp.float32)
  pcnt[...] += lax.dot_general(onehot, jnp.ones((blk, 1), jnp.float32),
                               (((0,), (0,)), ((), ())),
                               preferred_element_type=jnp.float32)

  lo = batch[0, 0, 0]
  hi = batch[0, blk - 1, 0]

  def mbody(g, carry):
    m = jnp.max(jnp.where(ids == g, h, -1e30), axis=0, keepdims=True)
    pmax[pl.ds(g, 1), :] = jnp.maximum(pmax[pl.ds(g, 1), :], m)
    return carry

  lax.fori_loop(lo, hi + 1, mbody, 0)

  @pl.when(i == nb - 1)
  def _():
    gcnt = pcnt[...]
    gmean = psum[...] / jnp.maximum(gcnt, 1.0)
    gmax = jnp.where(gcnt > 0, pmax[...], 0.0)
    pooled = jnp.concatenate([gmax, gmean], axis=1)
    out[...] = jnp.dot(pooled, w_out[...],
                       preferred_element_type=jnp.float32) + b_out[...]


def _row_spec(blk, width):
  return pl.BlockSpec((blk, width), lambda i: (i, 0))


def _full_spec(shape):
  return pl.BlockSpec(shape, lambda i: tuple(0 for _ in shape))


def kernel(x, edge_index, batch, W_l1, b_l1, W_r1, W_l2, b_l2, W_r2,
           W_out, b_out):
  n, d = x.shape
  e = edge_index.shape[1]

  per_tile = -(-e // (NS * CHUNK * SUPER)) * CHUNK * SUPER
  nchunks = per_tile // CHUNK
  e_pad = per_tile * NS
  src = jnp.concatenate([edge_index[0], jnp.zeros((e_pad - e,), jnp.int32)])
  dst = jnp.concatenate([edge_index[1],
                         jnp.full((e_pad - e,), N_NODES, jnp.int32)])
  src_base = src.reshape(NS, nchunks, CHUNK)
  src_r = jnp.stack([src_base, src_base + n])
  dst_r = jnp.broadcast_to(dst.reshape(1, NS, nchunks, CHUNK),
                           (NC, NS, nchunks, CHUNK))
  dst_w = dst.reshape(NC, NS, nchunks // NC, CHUNK)
  xcat = jnp.concatenate([x[:, :HALF], x[:, HALF:]], axis=0)
  zeros_rows = jnp.zeros((ZROWS, HALF), jnp.float32)
  ones_rows = jnp.ones((CHUNK, HALF), jnp.float32)

  sc_agg = _make_sc_aggregate(nchunks)
  sc_cnt = _make_sc_count(nchunks // NC)

  cntf = sc_cnt(dst_w, zeros_rows, ones_rows)
  cnt = cntf.reshape(NC, N_ACC, HALF)
  agg1 = sc_agg(xcat, src_r, dst_r, zeros_rows).reshape(NC, N_ACC, HALF)

  blk = 1000
  nb = n // blk
  half_spec = lambda k: pl.BlockSpec((1, blk, HALF), lambda i, k=k: (k, i, 0))

  h1 = pl.pallas_call(
      _sage_tc_kernel,
      grid=(nb,),
      in_specs=[
          half_spec(0), half_spec(1),
          _row_spec(blk, HALF),
          pl.BlockSpec((blk, HALF), lambda i: (nb + i, 0)),
          half_spec(0), half_spec(1),
          _full_spec((FDIM, FDIM)), _full_spec((1, FDIM)),
          _full_spec((FDIM, FDIM)),
      ],
      out_specs=pl.BlockSpec((2, blk, HALF), lambda i: (0, i, 0)),
      out_shape=jax.ShapeDtypeStruct((2, n, HALF), jnp.float32),
  )(agg1, agg1, xcat, xcat, cnt, cnt, W_l1, b_l1.reshape(1, FDIM), W_r1)

  agg2 = sc_agg(h1.reshape(2 * n, HALF), src_r, dst_r,
                zeros_rows).reshape(NC, N_ACC, HALF)

  batch3 = batch.reshape(nb, blk, 1)
  out = pl.pallas_call(
      functools.partial(_sage_pool_tc_kernel, nb, blk),
      grid=(nb,),
      in_specs=[
          half_spec(0), half_spec(1), half_spec(0), half_spec(1),
          half_spec(0), half_spec(1),
          _full_spec((FDIM, FDIM)), _full_spec((1, FDIM)),
          _full_spec((FDIM, FDIM)),
          pl.BlockSpec((1, blk, 1), lambda i: (i, 0, 0)),
          _full_spec((2 * FDIM, NGRAPH * 2)), _full_spec((1, NGRAPH * 2)),
      ],
      out_specs=pl.BlockSpec((NGRAPH, NGRAPH * 2), lambda i: (0, 0)),
      out_shape=jax.ShapeDtypeStruct((NGRAPH, NGRAPH * 2), jnp.float32),
      scratch_shapes=[
          pltpu.VMEM((NGRAPH, FDIM), jnp.float32),
          pltpu.VMEM((NGRAPH, FDIM), jnp.float32),
          pltpu.VMEM((NGRAPH, 1), jnp.float32),
      ],
  )(agg2, agg2, h1, h1, cnt, cnt, W_l2, b_l2.reshape(1, FDIM), W_r2,
    batch3, W_out, b_out.reshape(1, NGRAPH * 2))
  return out


# speedup vs baseline: 4.1791x; 1.1573x over previous
import functools

import jax
import jax.numpy as jnp
from jax import lax
from jax.experimental import pallas as pl
from jax.experimental.pallas import tpu as pltpu
from jax.experimental.pallas import tpu_sc as plsc

N_NODES = 10000
FDIM = 256
HALF = 128
NGRAPH = 64

NC = 2
NS = 16
CHUNK = 128
SUPER = 8

ZROWS = -(-(N_NODES + 1) // (NS * 8)) * 8
N_ACC = ZROWS * NS


def _sc_aggregate_kernel(nchunks, xcat, src_r, dst_r, zeros_rows,
                         agg_out, acc, src_v, dst_v, rows2,
                         gsem0, gsem1, ssem0, ssem1):
  c = lax.axis_index("c")
  s = lax.axis_index("s")
  gsem = (gsem0, gsem1)
  ssem = (ssem0, ssem1)

  pltpu.sync_copy(zeros_rows, acc.at[pl.ds(s * ZROWS, ZROWS)])
  plsc.subcore_barrier()

  def outer(u, carry):
    pltpu.sync_copy(src_r.at[c, s, pl.ds(u * SUPER, SUPER)], src_v)
    pltpu.sync_copy(dst_r.at[c, s, pl.ds(u * SUPER, SUPER)], dst_v)
    g = [None, None]
    sc = [None, None]
    g[0] = pltpu.async_copy(xcat.at[src_v.at[0]], rows2.at[0], gsem[0])
    for j in range(SUPER):
      b = j % 2
      if j + 1 < SUPER:
        if j >= 1:
          sc[1 - b].wait()
        g[1 - b] = pltpu.async_copy(xcat.at[src_v.at[j + 1]],
                                    rows2.at[1 - b], gsem[1 - b])
      g[b].wait()
      sc[b] = pltpu.async_copy(rows2.at[b], acc.at[dst_v.at[j]],
                               ssem[b], add=True)
    sc[0].wait()
    sc[1].wait()
    return carry

  lax.fori_loop(0, nchunks // SUPER, outer, 0)
  plsc.subcore_barrier()

  pltpu.sync_copy(acc.at[pl.ds(s * ZROWS, ZROWS)],
                  agg_out.at[pl.ds(c * N_ACC + s * ZROWS, ZROWS)])


def _make_sc_aggregate(nchunks):
  mesh = plsc.VectorSubcoreMesh(core_axis_name="c", subcore_axis_name="s")
  scratch = [
      pltpu.VMEM_SHARED((N_ACC, HALF), jnp.float32),
      pltpu.VMEM((SUPER, CHUNK), jnp.int32),
      pltpu.VMEM((SUPER, CHUNK), jnp.int32),
      pltpu.VMEM((2, CHUNK, HALF), jnp.float32),
      pltpu.SemaphoreType.DMA,
      pltpu.SemaphoreType.DMA,
      pltpu.SemaphoreType.DMA,
      pltpu.SemaphoreType.DMA,
  ]
  return pl.kernel(
      functools.partial(_sc_aggregate_kernel, nchunks),
      out_type=jax.ShapeDtypeStruct((NC * N_ACC, HALF), jnp.float32),
      mesh=mesh, scratch_types=scratch)


def _sc_count_kernel(nchunks, dst_r, zeros_rows, ones_rows,
                     cnt_out, acc, dst_v, ones_v, gsem):
  c = lax.axis_index("c")
  s = lax.axis_index("s")

  pltpu.sync_copy(zeros_rows, acc.at[pl.ds(s * ZROWS, ZROWS)])
  pltpu.sync_copy(ones_rows, ones_v)
  plsc.subcore_barrier()

  def outer(u, carry):
    pltpu.sync_copy(dst_r.at[c, s, pl.ds(u * SUPER, SUPER)], dst_v)
    ds = [pltpu.async_copy(ones_v, acc.at[dst_v.at[j]], gsem, add=True)
          for j in range(SUPER)]
    for d in ds:
      d.wait()
    return carry

  lax.fori_loop(0, nchunks // SUPER, outer, 0)
  plsc.subcore_barrier()

  pltpu.sync_copy(acc.at[pl.ds(s * ZROWS, ZROWS)],
                  cnt_out.at[pl.ds(c * N_ACC + s * ZROWS, ZROWS)])


def _make_sc_count(nchunks):
  mesh = plsc.VectorSubcoreMesh(core_axis_name="c", subcore_axis_name="s")
  scratch = [
      pltpu.VMEM_SHARED((N_ACC, HALF), jnp.float32),
      pltpu.VMEM((SUPER, CHUNK), jnp.int32),
      pltpu.VMEM((CHUNK, HALF), jnp.float32),
      pltpu.SemaphoreType.DMA,
  ]
  return pl.kernel(
      functools.partial(_sc_count_kernel, nchunks),
      out_type=jax.ShapeDtypeStruct((NC * N_ACC, HALF), jnp.float32),
      mesh=mesh, scratch_types=scratch)


def _sage_tc_kernel(agg_lo, agg_hi, x_lo, x_hi, cnt0, cnt1, w_l, b_l, w_r,
                    h_out):
  inv = 1.0 / jnp.maximum(cnt0[0, :, 0:1] + cnt1[0, :, 0:1], 1.0)
  acc = jnp.dot(agg_lo[0], w_l[:HALF, :], preferred_element_type=jnp.float32)
  acc += jnp.dot(agg_hi[0], w_l[HALF:, :], preferred_element_type=jnp.float32)
  acc *= inv
  acc += jnp.dot(x_lo[...], w_r[:HALF, :], preferred_element_type=jnp.float32)
  acc += jnp.dot(x_hi[...], w_r[HALF:, :], preferred_element_type=jnp.float32)
  acc += b_l[...]
  h = jnp.maximum(acc, 0.0)
  h_out[0] = h[:, :HALF]
  h_out[1] = h[:, HALF:]


def _sage_pool_tc_kernel(nb, blk, agg_lo, agg_hi, x_lo, x_hi, cnt0, cnt1,
                         w_l, b_l, w_r, batch, w_out, b_out, out,
                         psum, pmax, pcnt):
  i = pl.program_id(0)

  @pl.when(i == 0)
  def _():
    psum[...] = jnp.zeros_like(psum)
    pmax[...] = jnp.full_like(pmax, -1e30)
    pcnt[...] = jnp.zeros_like(pcnt)

  inv = 1.0 / jnp.maximum(cnt0[0, :, 0:1] + cnt1[0, :, 0:1], 1.0)
  acc = jnp.dot(agg_lo[0], w_l[:HALF, :], preferred_element_type=jnp.float32)
  acc += jnp.dot(agg_hi[0], w_l[HALF:, :], preferred_element_type=jnp.float32)
  acc *= inv
  acc += jnp.dot(x_lo[0], w_r[:HALF, :], preferred_element_type=jnp.float32)
  acc += jnp.dot(x_hi[0], w_r[HALF:, :], preferred_element_type=jnp.float32)
  acc += b_l[...]
  h = jnp.maximum(acc, 0.0)

  ids = batch[0]
  onehot = (ids == lax.broadcasted_iota(jnp.int32, (blk, NGRAPH), 1)
            ).astype(jnp.float32)
  psum[...] += lax.dot_general(onehot, h, (((0,), (0,)), ((), ())),
                               preferred_element_type=jnp.float32)
  pcnt[...] += lax.dot_general(onehot, jnp.ones((blk, 1), jnp.float32),
                               (((0,), (0,)), ((), ())),
                               preferred_element_type=jnp.float32)

  lo = batch[0, 0, 0]
  hi = batch[0, blk - 1, 0]

  def mbody(g, carry):
    m = jnp.max(jnp.where(ids == g, h, -1e30), axis=0, keepdims=True)
    pmax[pl.ds(g, 1), :] = jnp.maximum(pmax[pl.ds(g, 1), :], m)
    return carry

  lax.fori_loop(lo, hi + 1, mbody, 0)

  @pl.when(i == nb - 1)
  def _():
    gcnt = pcnt[...]
    gmean = psum[...] / jnp.maximum(gcnt, 1.0)
    gmax = jnp.where(gcnt > 0, pmax[...], 0.0)
    pooled = jnp.concatenate([gmax, gmean], axis=1)
    out[...] = jnp.dot(pooled, w_out[...],
                       preferred_element_type=jnp.float32) + b_out[...]


def _row_spec(blk, width):
  return pl.BlockSpec((blk, width), lambda i: (i, 0))


def _full_spec(shape):
  return pl.BlockSpec(shape, lambda i: tuple(0 for _ in shape))


def kernel(x, edge_index, batch, W_l1, b_l1, W_r1, W_l2, b_l2, W_r2,
           W_out, b_out):
  n, d = x.shape
  e = edge_index.shape[1]

  per_tile = -(-e // (NS * CHUNK * SUPER)) * CHUNK * SUPER
  nchunks = per_tile // CHUNK
  e_pad = per_tile * NS
  src = jnp.concatenate([edge_index[0], jnp.zeros((e_pad - e,), jnp.int32)])
  dst = jnp.concatenate([edge_index[1],
                         jnp.full((e_pad - e,), N_NODES, jnp.int32)])
  src_base = src.reshape(NS, nchunks, CHUNK)
  src_r = jnp.stack([src_base, src_base + n])
  dst_r = jnp.broadcast_to(dst.reshape(1, NS, nchunks, CHUNK),
                           (NC, NS, nchunks, CHUNK))
  dst_w = dst.reshape(NC, NS, nchunks // NC, CHUNK)
  xcat = jnp.concatenate([x[:, :HALF], x[:, HALF:]], axis=0)
  zeros_rows = jnp.zeros((ZROWS, HALF), jnp.float32)
  ones_rows = jnp.ones((CHUNK, HALF), jnp.float32)

  sc_agg = _make_sc_aggregate(nchunks)
  sc_cnt = _make_sc_count(nchunks // NC)

  cntf = sc_cnt(dst_w, zeros_rows, ones_rows)
  cnt = cntf.reshape(NC, N_ACC, HALF)
  agg1 = sc_agg(xcat, src_r, dst_r, zeros_rows).reshape(NC, N_ACC, HALF)

  blk = 1000
  nb = n // blk
  half_spec = lambda k: pl.BlockSpec((1, blk, HALF), lambda i, k=k: (k, i, 0))

  h1 = pl.pallas_call(
      _sage_tc_kernel,
      grid=(nb,),
      in_specs=[
          half_spec(0), half_spec(1),
          _row_spec(blk, HALF),
          pl.BlockSpec((blk, HALF), lambda i: (nb + i, 0)),
          half_spec(0), half_spec(1),
          _full_spec((FDIM, FDIM)), _full_spec((1, FDIM)),
          _full_spec((FDIM, FDIM)),
      ],
      out_specs=pl.BlockSpec((2, blk, HALF), lambda i: (0, i, 0)),
      out_shape=jax.ShapeDtypeStruct((2, n, HALF), jnp.float32),
  )(agg1, agg1, xcat, xcat, cnt, cnt, W_l1, b_l1.reshape(1, FDIM), W_r1)

  agg2 = sc_agg(h1.reshape(2 * n, HALF), src_r, dst_r,
                zeros_rows).reshape(NC, N_ACC, HALF)

  batch3 = batch.reshape(nb, blk, 1)
  out = pl.pallas_call(
      functools.partial(_sage_pool_tc_kernel, nb, blk),
      grid=(nb,),
      in_specs=[
          half_spec(0), half_spec(1), half_spec(0), half_spec(1),
          half_spec(0), half_spec(1),
          _full_spec((FDIM, FDIM)), _full_spec((1, FDIM)),
          _full_spec((FDIM, FDIM)),
          pl.BlockSpec((1, blk, 1), lambda i: (i, 0, 0)),
          _full_spec((2 * FDIM, NGRAPH * 2)), _full_spec((1, NGRAPH * 2)),
      ],
      out_specs=pl.BlockSpec((NGRAPH, NGRAPH * 2), lambda i: (0, 0)),
      out_shape=jax.ShapeDtypeStruct((NGRAPH, NGRAPH * 2), jnp.float32),
      scratch_shapes=[
          pltpu.VMEM((NGRAPH, FDIM), jnp.float32),
          pltpu.VMEM((NGRAPH, FDIM), jnp.float32),
          pltpu.VMEM((NGRAPH, 1), jnp.float32),
      ],
  )(agg2, agg2, h1, h1, cnt, cnt, W_l2, b_l2.reshape(1, FDIM), W_r2,
    batch3, W_out, b_out.reshape(1, NGRAPH * 2))
  return out
